# baseline (speedup 1.0000x reference)
import functools

import jax
import jax.numpy as jnp
from jax import lax
from jax.experimental import pallas as pl
from jax.experimental.pallas import tpu as pltpu
from jax.experimental.pallas import tpu_sc as plsc

_SIZES = (119, 5, 12, 12, 10, 8)
_D = 64
_N = 100000
_NC, _NS = 2, 16
_NW = _NC * _NS
_CHUNK = 3136
_TAIL = _N - (_NW - 1) * _CHUNK
_VPW = _CHUNK // 16
_VPT = _TAIL // 16
_TOFF = tuple(sum(s * _D for s in _SIZES[:i]) for i in range(6))
_TTOT = sum(s * _D for s in _SIZES)
_REP = 129
_GF = _REP * 16


def _sc_body(tcat_hbm, f0, f1, f2, f3, f4, f5, out_hbm,
             t_v, g_v, c0, c1, c2, c3, c4, c5, acc_v, *sems):
    wid = lax.axis_index("s") * _NC + lax.axis_index("c")
    frefs = (f0, f1, f2, f3, f4, f5)
    chunks = (c0, c1, c2, c3, c4, c5)
    zeros = jnp.zeros((16,), jnp.float32)

    copies = []
    for fref, chunk, sem in zip(frefs, chunks, sems):
        cf = pltpu.make_async_copy(fref.at[pl.ds(wid * _CHUNK, _CHUNK)],
                                   chunk.at[pl.ds(0, _CHUNK)], sem)
        ct = pltpu.make_async_copy(fref.at[pl.ds(wid * _CHUNK, _TAIL)],
                                   chunk.at[pl.ds(0, _TAIL)], sem)
        copies.append((cf, ct))

        @pl.when(wid < _NW - 1)
        def _(cf=cf):
            cf.start()

        @pl.when(wid == _NW - 1)
        def _(ct=ct):
            ct.start()

    pltpu.sync_copy(tcat_hbm, t_v)

    lane = lax.iota(jnp.int32, 16)
    for fi, size in enumerate(_SIZES):
        for j in range((size + 15) // 16):
            base = _TOFF[fi] + j * 16 + lane

            def dbody(d, accs, base=base, size=size):
                return tuple(
                    a + plsc.load_gather(t_v, [base + (4 * d + k) * size])
                    for k, a in enumerate(accs))

            accs = lax.fori_loop(0, _D // 4, dbody, (zeros,) * 4, unroll=2)
            gvec = sum(accs)
            for c in range(16):
                g_v[pl.ds(_GF * fi + _REP * c + 16 * j, 16)] = gvec

    @pl.when(wid == _NW - 1)
    def _():
        for chunk in chunks:
            for j in range(_VPT, _VPW):
                chunk[pl.ds(j * 16, 16)] = zeros

    lanebias = lane * _REP
    acc_f = zeros
    for fi, (chunk, (cf, ct)) in enumerate(zip(chunks, copies)):
        @pl.when(wid < _NW - 1)
        def _(cf=cf):
            cf.wait()

        @pl.when(wid == _NW - 1)
        def _(ct=ct):
            ct.wait()

        gseg = g_v.at[pl.ds(_GF * fi, _GF)]

        def body(i, accs, chunk=chunk, gseg=gseg):
            out = []
            for k, a in enumerate(accs):
                x = chunk[pl.ds((4 * i + k) * 16, 16)]
                out.append(a + x * plsc.load_gather(
                    gseg, [x.astype(jnp.int32) + lanebias]))
            return tuple(out)

        accs = lax.fori_loop(0, _VPW // 4, body, (zeros,) * 4, unroll=2)
        acc_f = acc_f + (accs[0] + accs[1]) + (accs[2] + accs[3])
    acc_v[...] = acc_f
    pltpu.sync_copy(acc_v, out_hbm.at[pl.ds(wid * 16, 16)])


@functools.cache
def _sc_call():
    return functools.partial(
        pl.kernel,
        out_type=jax.ShapeDtypeStruct((_NW * 16,), jnp.float32),
        mesh=plsc.VectorSubcoreMesh(core_axis_name="c", subcore_axis_name="s",
                                    num_cores=_NC, num_subcores=_NS),
        compiler_params=pltpu.CompilerParams(needs_layout_passes=False,
                                             use_tc_tiling_on_sc=False),
        scratch_types=[
            pltpu.VMEM((_TTOT,), jnp.float32),
            pltpu.VMEM((_GF * 6,), jnp.float32),
        ] + [pltpu.VMEM((_CHUNK,), jnp.float32)] * 6 + [
            pltpu.VMEM((16,), jnp.float32),
        ] + [pltpu.SemaphoreType.DMA] * 6,
    )(_sc_body)


def kernel(atomic_num, chirality, degree, formal_charge, num_h, hybridization,
           table_atomic_num, table_chirality, table_degree, table_formal_charge,
           table_num_h, table_hybridization):
    feats = (atomic_num, chirality, degree, formal_charge, num_h, hybridization)
    tables = (table_atomic_num, table_chirality, table_degree,
              table_formal_charge, table_num_h, table_hybridization)
    tcat = (jnp.concatenate([t.T.reshape(-1) for t in tables])
            .astype(jnp.bfloat16).astype(jnp.float32))
    partials = _sc_call()(tcat, *feats)
    return jnp.sum(partials)
